# baseline (speedup 1.0000x reference)
import functools

import jax
import jax.numpy as jnp
from jax import lax
from jax.experimental import pallas as pl
from jax.experimental.pallas import tpu as pltpu
from jax.experimental.pallas import tpu_sc as plsc

_CLASSES = 100000
_FEAT = 64
_BATCH = 16384

_NC = 2
_NS = 16
_NW = _NC * _NS
_BPW = _BATCH // _NW
_ORPW = _BPW // 128
_HRPS = (_BATCH // _NS) // 128
_CPAD = 102400
_ZCH = _CPAD // _NS


def _sc_gather_and_counts(label2d, centers):
    mesh = plsc.VectorSubcoreMesh(core_axis_name="c", subcore_axis_name="s")

    @functools.partial(
        pl.kernel,
        out_type=(
            jax.ShapeDtypeStruct((_BATCH, _FEAT), jnp.float32),
            jax.ShapeDtypeStruct((_BATCH,), jnp.float32),
        ),
        mesh=mesh,
        compiler_params=pltpu.CompilerParams(use_tc_tiling_on_sc=False),
        scratch_types=[
            pltpu.VMEM_SHARED((_CPAD,), jnp.float32),
            pltpu.VMEM((_HRPS, 128), jnp.int32),
            pltpu.VMEM((_ORPW, 128), jnp.int32),
            pltpu.VMEM((128,), jnp.float32),
            pltpu.VMEM((_ZCH,), jnp.float32),
            pltpu.VMEM((_BPW, _FEAT), jnp.float32),
            pltpu.VMEM((_BPW,), jnp.float32),
            pltpu.SemaphoreType.DMA,
        ],
    )
    def k(label_hbm, centers_hbm, scent_hbm, scnt_hbm,
          counts_sp, lab_h, lab_o, ones_v, zeros_v, rows_v, cnt_v, sem):
        c = lax.axis_index("c")
        s = lax.axis_index("s")
        wid = s * _NC + c

        pltpu.sync_copy(label_hbm.at[pl.ds(wid * _ORPW, _ORPW)], lab_o)
        gathers = []
        for j in range(_ORPW):
            gathers.append(pltpu.async_copy(
                centers_hbm.at[lab_o.at[j]],
                rows_v.at[pl.ds(j * 128, 128)], sem))

        @pl.loop(0, _ZCH, step=16)
        def _(i):
            zeros_v[pl.ds(i, 16)] = jnp.zeros((16,), jnp.float32)

        @pl.loop(0, 128, step=16)
        def _(i):
            ones_v[pl.ds(i, 16)] = jnp.full((16,), 1.0, jnp.float32)

        pltpu.sync_copy(zeros_v, counts_sp.at[pl.ds(s * _ZCH, _ZCH)])
        plsc.subcore_barrier()

        pltpu.sync_copy(label_hbm.at[pl.ds(s * _HRPS, _HRPS)], lab_h)
        for j in range(_HRPS):
            pltpu.sync_copy(ones_v, counts_sp.at[lab_h.at[j]], add=True)
        plsc.subcore_barrier()

        for j in range(_ORPW):
            pltpu.sync_copy(counts_sp.at[lab_o.at[j]],
                            cnt_v.at[pl.ds(j * 128, 128)])

        for g in gathers:
            g.wait()
        pltpu.sync_copy(rows_v, scent_hbm.at[pl.ds(wid * _BPW, _BPW)])
        pltpu.sync_copy(cnt_v, scnt_hbm.at[pl.ds(wid * _BPW, _BPW)])

    return k(label2d, centers)


def _tc_loss(feat, scent, scnt):
    def body(feat_ref, scent_ref, cnt_ref, out_ref):
        d = feat_ref[...] - scent_ref[...]
        ss = jnp.sum(d * d, axis=1, keepdims=True)
        r = ss / (cnt_ref[...] + 1.0)
        out_ref[0, 0] = jnp.sum(jnp.sqrt(r))

    out = pl.pallas_call(
        body,
        out_shape=jax.ShapeDtypeStruct((1, 1), jnp.float32),
        out_specs=pl.BlockSpec(memory_space=pltpu.SMEM),
    )(feat, scent, scnt.reshape(_BATCH, 1))
    return out[0, 0]


def kernel(feat, label, centers):
    label2d = label.reshape(_BATCH // 128, 128)
    scent, scnt = _sc_gather_and_counts(label2d, centers)
    return _tc_loss(feat, scent, scnt)


# speedup vs baseline: 1.0538x; 1.0538x over previous
import functools

import jax
import jax.numpy as jnp
from jax import lax
from jax.experimental import pallas as pl
from jax.experimental.pallas import tpu as pltpu
from jax.experimental.pallas import tpu_sc as plsc

_CLASSES = 100000
_FEAT = 64
_BATCH = 16384

_NC = 2
_NS = 16
_NW = _NC * _NS
_BPW = _BATCH // _NW
_ORPW = _BPW // 128
_HRPS = (_BATCH // _NS) // 128
_CPAD = 102400
_ZCH = _CPAD // _NS
_BLK = 16


def _sc_partials(label2d, feat128, pairs):
    mesh = plsc.VectorSubcoreMesh(core_axis_name="c", subcore_axis_name="s")

    @functools.partial(
        pl.kernel,
        out_type=(
            jax.ShapeDtypeStruct((_BATCH // 8, 128), jnp.float32),
            jax.ShapeDtypeStruct((_BATCH,), jnp.float32),
        ),
        mesh=mesh,
        scratch_types=[
            pltpu.VMEM_SHARED((_CPAD,), jnp.float32),
            pltpu.VMEM((_HRPS, 128), jnp.int32),
            pltpu.VMEM((_ORPW, 128), jnp.int32),
            pltpu.VMEM((_ORPW, 128), jnp.int32),
            pltpu.VMEM((128,), jnp.float32),
            pltpu.VMEM((_ZCH,), jnp.float32),
            pltpu.VMEM((_BPW, 128), jnp.float32),
            pltpu.VMEM((_BPW // 2, 128), jnp.float32),
            pltpu.VMEM((_BPW // 8, 128), jnp.float32),
            pltpu.VMEM((_BPW,), jnp.float32),
            pltpu.SemaphoreType.DMA,
            pltpu.SemaphoreType.DMA,
            pltpu.SemaphoreType.DMA,
        ],
    )
    def k(label_hbm, feat_hbm, pairs_hbm, part_hbm, scnt_hbm,
          counts_sp, lab_h, lab_o, idx2_v, ones_v, zeros_v, rows_v, feat_v,
          part_v, cnt_v, sem_g, sem_f, sem_c):
        c = lax.axis_index("c")
        s = lax.axis_index("s")
        wid = s * _NC + c

        pltpu.sync_copy(label_hbm.at[pl.ds(wid * _ORPW, _ORPW)], lab_o)
        for j in range(_ORPW):
            for t in range(8):
                sl = pl.ds(t * 16, 16)
                idx2_v[j, sl] = lax.shift_right_logical(lab_o[j, sl], 1)
        gathers = []
        for j in range(_ORPW):
            gathers.append(pltpu.async_copy(
                pairs_hbm.at[idx2_v.at[j]],
                rows_v.at[pl.ds(j * 128, 128)], sem_g))
        feat_cp = pltpu.async_copy(
            feat_hbm.at[pl.ds(wid * (_BPW // 2), _BPW // 2)], feat_v, sem_f)

        @pl.loop(0, _ZCH, step=16)
        def _(i):
            zeros_v[pl.ds(i, 16)] = jnp.zeros((16,), jnp.float32)

        @pl.loop(0, 128, step=16)
        def _(i):
            ones_v[pl.ds(i, 16)] = jnp.full((16,), 1.0, jnp.float32)

        pltpu.sync_copy(zeros_v, counts_sp.at[pl.ds(s * _ZCH, _ZCH)])
        plsc.subcore_barrier()

        pltpu.sync_copy(label_hbm.at[pl.ds(s * _HRPS, _HRPS)], lab_h)
        for j in range(_HRPS):
            pltpu.sync_copy(ones_v, counts_sp.at[lab_h.at[j]], add=True)
        plsc.subcore_barrier()

        cgath = []
        for j in range(_ORPW):
            cgath.append(pltpu.async_copy(
                counts_sp.at[lab_o.at[j]],
                cnt_v.at[pl.ds(j * 128, 128)], sem_c))

        feat_cp.wait()
        for g in gathers:
            g.wait()

        @pl.loop(0, _BPW, step=_BLK)
        def _(b):
            lv = lab_o[b // 128, pl.ds(b % 128, 16)]
            for ii in range(_BLK):
                off = (lv[ii] & 1) * _FEAT
                acc = jnp.zeros((16,), jnp.float32)
                for kk in range(_FEAT // 16):
                    f = feat_v[b // 2 + ii // 2,
                               pl.ds((ii % 2) * _FEAT + kk * 16, 16)]
                    g = rows_v[b + ii, pl.ds(off + kk * 16, 16)]
                    d = f - g
                    acc = acc + d * d
                part_v[b // 8 + ii // 8, pl.ds((ii % 8) * 16, 16)] = acc

        pltpu.sync_copy(part_v, part_hbm.at[pl.ds(wid * (_BPW // 8), _BPW // 8)])
        for g in cgath:
            g.wait()
        pltpu.sync_copy(cnt_v, scnt_hbm.at[pl.ds(wid * _BPW, _BPW)])

    return k(label2d, feat128, pairs)


def _tc_loss(part, scnt):
    def body(p_ref, c_ref, out_ref):
        p = p_ref[...]
        i0 = lax.broadcasted_iota(jnp.int32, (_BATCH // 8, 128), 0)
        i1 = lax.broadcasted_iota(jnp.int32, (_BATCH // 8, 128), 1)
        m = (i0 // 16 == i1).astype(jnp.float32)
        ss = jax.lax.dot(p, m, precision=jax.lax.Precision.HIGHEST)
        r = ss / (c_ref[...] + 1.0)
        out_ref[0, 0] = jnp.sum(jnp.sqrt(r))

    out = pl.pallas_call(
        body,
        out_shape=jax.ShapeDtypeStruct((1, 1), jnp.float32),
        out_specs=pl.BlockSpec(memory_space=pltpu.SMEM),
    )(part.reshape(128, _BATCH // 8), scnt.reshape(128, 128))
    return out[0, 0]


def kernel(feat, label, centers):
    label2d = label.reshape(_BATCH // 128, 128)
    feat128 = feat.reshape(_BATCH // 2, 128)
    pairs = centers.reshape(_CLASSES // 2, 128)
    part, scnt = _sc_partials(label2d, feat128, pairs)
    return _tc_loss(part, scnt)
